# baseline (speedup 1.0000x reference)
import jax
import jax.numpy as jnp
from jax.experimental import pallas as pl
from jax.experimental.pallas import tpu as pltpu

N = 2073600
Q = N // 8
FEAT_DIM = 12
K = 32
BLKQ = 5760


def _fused_body(ft_ref, rt_ref, cb_ref, cbw1t_ref, w1rt_ref, b1_ref,
                w2t_ref, b2_ref, w3t_ref, b3_ref, out_ref):
    cb = cb_ref[...]
    cb_half_sq = 0.5 * jnp.sum(cb * cb, axis=1)[:, None]
    ft = ft_ref[...]
    rt = rt_ref[...]

    outs = []
    for a in range(8):
        fa = ft[12 * a:12 * a + 12, :]
        ra = rt[3 * a:3 * a + 3, :]

        scores = jnp.dot(cb, fa, preferred_element_type=jnp.float32) - cb_half_sq
        m = jnp.max(scores, axis=0, keepdims=True)
        ii = jax.lax.broadcasted_iota(jnp.int32, scores.shape, 0)
        masked_ii = jnp.where(scores >= m, ii, K)
        amin = jnp.min(masked_ii, axis=0, keepdims=True)
        one_hot = (ii == amin).astype(jnp.float32)

        h = (jnp.dot(cbw1t_ref[...], one_hot, preferred_element_type=jnp.float32)
             + jnp.dot(w1rt_ref[...], ra, preferred_element_type=jnp.float32)
             + b1_ref[...])
        h = jnp.maximum(h, 0.0)
        h = jnp.dot(w2t_ref[...], h, preferred_element_type=jnp.float32) + b2_ref[...]
        h = jnp.maximum(h, 0.0)
        o = jnp.dot(w3t_ref[...], h, preferred_element_type=jnp.float32) + b3_ref[...]
        outs.append(jnp.clip(jax.nn.sigmoid(o), 0.0, 1.0))

    out_ref[...] = jnp.concatenate(outs, axis=0)


@jax.jit
def _run(ft, rt, codebook, cbw1t, w1rt, b1, w2t, b2, w3t, b3):
    rep = lambda shape: pl.BlockSpec(shape, lambda i: (0, 0))
    return pl.pallas_call(
        _fused_body,
        grid=(Q // BLKQ,),
        in_specs=[
            pl.BlockSpec((96, BLKQ), lambda i: (0, i)),
            pl.BlockSpec((24, BLKQ), lambda i: (0, i)),
            rep((K, FEAT_DIM)),
            rep((K, K)),
            rep((K, 3)),
            rep((K, 1)),
            rep((K, K)),
            rep((K, 1)),
            rep((3, K)),
            rep((3, 1)),
        ],
        out_specs=pl.BlockSpec((24, BLKQ), lambda i: (0, i)),
        out_shape=jax.ShapeDtypeStruct((24, Q), jnp.float32),
        compiler_params=pltpu.CompilerParams(
            dimension_semantics=("arbitrary",),
        ),
    )(ft, rt, codebook, cbw1t, w1rt, b1, w2t, b2, w3t, b3)


def kernel(feat_enc, rays_d, codebook, W1, b1, W2, b2, W3, b3):
    ft = feat_enc.reshape(Q, 96).T
    rt = rays_d.reshape(Q, 24).T
    cbw1t = (codebook @ W1[:FEAT_DIM]).T
    out_t = _run(ft, rt, codebook, cbw1t, W1[FEAT_DIM:].T,
                 b1.reshape(K, 1), W2.T, b2.reshape(K, 1),
                 W3.T, b3.reshape(3, 1))
    return out_t.T.reshape(N, 3)
